# Initial kernel scaffold; baseline (speedup 1.0000x reference)
import jax
import jax.numpy as jnp
from jax.experimental import pallas as pl


def kernel(x, edge_index, W1, b1, W2, b2):
    raise NotImplementedError("write your pallas kernel here")


# speedup vs baseline: 16.5093x; 16.5093x over previous
import functools

import jax
import jax.numpy as jnp
from jax import lax
from jax.experimental import pallas as pl
from jax.experimental.pallas import tpu as pltpu
from jax.experimental.pallas import tpu_sc as plsc

N = 10000
E = 320000
D = 128

NC = 2
NS = 16
NW = NC * NS

CHUNK = 128
ECH = E // CHUNK
ECH_C = ECH // NC
ECH_T = (ECH_C + NS - 1) // NS
NP = 10240
PBLK = NP // CHUNK
RBM = 80
RBLOCKS = N // RBM

assert ECH * CHUNK == E and ECH_C * NC == ECH and RBLOCKS * RBM == N
assert (E // NC) % CHUNK == 0

_mesh = plsc.VectorSubcoreMesh(core_axis_name="c", subcore_axis_name="s")


def _fill_zero_rows(buf):
    @pl.loop(0, CHUNK)
    def _(i):
        for j in range(D // 16):
            buf[i, pl.ds(j * 16, 16)] = jnp.zeros((16,), jnp.float32)


@functools.partial(
    pl.kernel,
    out_type=jax.ShapeDtypeStruct((NC, 1, NP), jnp.float32),
    mesh=_mesh,
    scratch_types=[
        pltpu.VMEM((CHUNK,), jnp.int32),
        pltpu.VMEM((CHUNK,), jnp.float32),
        pltpu.VMEM((CHUNK,), jnp.float32),
        pltpu.VMEM_SHARED((NP,), jnp.float32),
    ],
)
def _degree_kernel(dst_hbm, out_hbm, idx_v, ones_v, zero_v, acc_sh):
    c = lax.axis_index("c")
    s = lax.axis_index("s")
    for k in range(CHUNK // 16):
        ones_v[pl.ds(k * 16, 16)] = jnp.ones((16,), jnp.float32)
        zero_v[pl.ds(k * 16, 16)] = jnp.zeros((16,), jnp.float32)

    @pl.loop(0, PBLK // NS)
    def _(k):
        b = s + k * NS
        pltpu.sync_copy(zero_v, acc_sh.at[pl.ds(b * CHUNK, CHUNK)])

    plsc.subcore_barrier()

    ebase = c * (E // NC)

    @pl.loop(0, ECH_T)
    def _(k):
        q = s + k * NS

        @pl.when(q < ECH_C)
        def _():
            pltpu.sync_copy(dst_hbm.at[pl.ds(ebase + q * CHUNK, CHUNK)], idx_v)
            pltpu.sync_copy(ones_v, acc_sh.at[idx_v], add=True)

    plsc.subcore_barrier()

    @pl.loop(0, PBLK // NS)
    def _(k):
        b = s + k * NS
        sl = pl.ds(b * CHUNK, CHUNK)
        pltpu.sync_copy(acc_sh.at[sl], out_hbm.at[c, 0, sl])


@functools.partial(
    pl.kernel,
    out_type=jax.ShapeDtypeStruct((NC, N, D), jnp.float32),
    mesh=_mesh,
    scratch_types=[
        pltpu.VMEM((CHUNK,), jnp.int32),
        pltpu.VMEM((CHUNK,), jnp.int32),
        pltpu.VMEM((CHUNK, D), jnp.float32),
        pltpu.VMEM_SHARED((N, D), jnp.float32),
        pltpu.SemaphoreType.DMA,
    ],
)
def _aggregate_kernel(g_hbm, src_hbm, dst_hbm, out_hbm, src_v, dst_v, rows_v,
                      acc_sh, sem):
    c = lax.axis_index("c")
    s = lax.axis_index("s")

    _fill_zero_rows(rows_v)

    @pl.loop(0, (RBLOCKS + NS - 1) // NS)
    def _(k):
        b = s + k * NS

        @pl.when(b < RBLOCKS)
        def _():
            pltpu.sync_copy(rows_v.at[pl.ds(0, RBM)],
                            acc_sh.at[pl.ds(b * RBM, RBM)])

    plsc.subcore_barrier()

    ebase = c * (E // NC)

    @pl.loop(0, ECH_T)
    def _(k):
        q = s + k * NS

        @pl.when(q < ECH_C)
        def _():
            off = ebase + q * CHUNK
            pltpu.sync_copy(src_hbm.at[pl.ds(off, CHUNK)], src_v)
            pltpu.sync_copy(dst_hbm.at[pl.ds(off, CHUNK)], dst_v)
            pltpu.async_copy(g_hbm.at[src_v], rows_v, sem).wait()
            pltpu.sync_copy(rows_v, acc_sh.at[dst_v], add=True)

    plsc.subcore_barrier()

    @pl.loop(0, (RBLOCKS + NS - 1) // NS)
    def _(k):
        b = s + k * NS

        @pl.when(b < RBLOCKS)
        def _():
            sl = pl.ds(b * RBM, RBM)
            pltpu.sync_copy(acc_sh.at[sl], out_hbm.at[c, sl])


BM = 2000
GRID = N // BM


def _mm_scale_body(x_ref, w_ref, dinv_ref, h_ref, g_ref):
    h = jnp.dot(x_ref[...], w_ref[...], preferred_element_type=jnp.float32)
    h_ref[...] = h
    g_ref[...] = h * dinv_ref[...]


_mm_scale = pl.pallas_call(
    _mm_scale_body,
    grid=(GRID,),
    in_specs=[
        pl.BlockSpec((BM, D), lambda i: (i, 0)),
        pl.BlockSpec((D, D), lambda i: (0, 0)),
        pl.BlockSpec((BM, 1), lambda i: (i, 0)),
    ],
    out_specs=[
        pl.BlockSpec((BM, D), lambda i: (i, 0)),
        pl.BlockSpec((BM, D), lambda i: (i, 0)),
    ],
    out_shape=[
        jax.ShapeDtypeStruct((N, D), jnp.float32),
        jax.ShapeDtypeStruct((N, D), jnp.float32),
    ],
)


def _combine_mm_body(agg_ref, h_ref, dinv_ref, b_ref, w_ref, h2_ref, g2_ref):
    dinv = dinv_ref[...]
    tot = agg_ref[0] + agg_ref[1]
    out1 = jnp.maximum(dinv * tot + (dinv * dinv) * h_ref[...] + b_ref[...], 0.0)
    h2 = jnp.dot(out1, w_ref[...], preferred_element_type=jnp.float32)
    h2_ref[...] = h2
    g2_ref[...] = h2 * dinv


_combine_mm = pl.pallas_call(
    _combine_mm_body,
    grid=(GRID,),
    in_specs=[
        pl.BlockSpec((NC, BM, D), lambda i: (0, i, 0)),
        pl.BlockSpec((BM, D), lambda i: (i, 0)),
        pl.BlockSpec((BM, 1), lambda i: (i, 0)),
        pl.BlockSpec((1, D), lambda i: (0, 0)),
        pl.BlockSpec((D, D), lambda i: (0, 0)),
    ],
    out_specs=[
        pl.BlockSpec((BM, D), lambda i: (i, 0)),
        pl.BlockSpec((BM, D), lambda i: (i, 0)),
    ],
    out_shape=[
        jax.ShapeDtypeStruct((N, D), jnp.float32),
        jax.ShapeDtypeStruct((N, D), jnp.float32),
    ],
)


def _combine_body(agg_ref, h_ref, dinv_ref, b_ref, out_ref):
    dinv = dinv_ref[...]
    tot = agg_ref[0] + agg_ref[1]
    out_ref[...] = dinv * tot + (dinv * dinv) * h_ref[...] + b_ref[...]


_combine = pl.pallas_call(
    _combine_body,
    grid=(GRID,),
    in_specs=[
        pl.BlockSpec((NC, BM, D), lambda i: (0, i, 0)),
        pl.BlockSpec((BM, D), lambda i: (i, 0)),
        pl.BlockSpec((BM, 1), lambda i: (i, 0)),
        pl.BlockSpec((1, D), lambda i: (0, 0)),
    ],
    out_specs=pl.BlockSpec((BM, D), lambda i: (i, 0)),
    out_shape=jax.ShapeDtypeStruct((N, D), jnp.float32),
)


def kernel(x, edge_index, W1, b1, W2, b2):
    src = edge_index[0]
    dst = edge_index[1]

    deg_part = _degree_kernel(dst)
    deg = deg_part[0, 0, :N] + deg_part[1, 0, :N] + 1.0
    dinv = lax.rsqrt(jnp.maximum(deg, 1.0))[:, None]

    h1, g1 = _mm_scale(x, W1, dinv)
    agg1 = _aggregate_kernel(g1, src, dst)
    h2, g2 = _combine_mm(agg1, h1, dinv, b1[None, :], W2)
    agg2 = _aggregate_kernel(g2, src, dst)
    out = _combine(agg2, h2, dinv, b2[None, :])
    return out


# speedup vs baseline: 23.9489x; 1.4506x over previous
import functools

import jax
import jax.numpy as jnp
from jax import lax
from jax.experimental import pallas as pl
from jax.experimental.pallas import tpu as pltpu
from jax.experimental.pallas import tpu_sc as plsc

N = 10000
E = 320000
D = 128

NC = 2
NS = 16
NW = NC * NS

CHUNK = 128
ECH = E // CHUNK
ECH_C = ECH // NC
ECH_T = (ECH_C + NS - 1) // NS
NP = 10240
PBLK = NP // CHUNK
RBM = 80
RBLOCKS = N // RBM

assert ECH * CHUNK == E and ECH_C * NC == ECH and RBLOCKS * RBM == N
assert (E // NC) % CHUNK == 0

_mesh = plsc.VectorSubcoreMesh(core_axis_name="c", subcore_axis_name="s")


def _fill_zero_rows(buf):
    @pl.loop(0, CHUNK)
    def _(i):
        for j in range(D // 16):
            buf[i, pl.ds(j * 16, 16)] = jnp.zeros((16,), jnp.float32)


@functools.partial(
    pl.kernel,
    out_type=jax.ShapeDtypeStruct((NC, 1, NP), jnp.float32),
    mesh=_mesh,
    scratch_types=[
        pltpu.VMEM((CHUNK,), jnp.int32),
        pltpu.VMEM((CHUNK,), jnp.float32),
        pltpu.VMEM((CHUNK,), jnp.float32),
        pltpu.VMEM_SHARED((NP,), jnp.float32),
    ],
)
def _degree_kernel(dst_hbm, out_hbm, idx_v, ones_v, zero_v, acc_sh):
    c = lax.axis_index("c")
    s = lax.axis_index("s")
    for k in range(CHUNK // 16):
        ones_v[pl.ds(k * 16, 16)] = jnp.ones((16,), jnp.float32)
        zero_v[pl.ds(k * 16, 16)] = jnp.zeros((16,), jnp.float32)

    @pl.loop(0, PBLK // NS)
    def _(k):
        b = s + k * NS
        pltpu.sync_copy(zero_v, acc_sh.at[pl.ds(b * CHUNK, CHUNK)])

    plsc.subcore_barrier()

    ebase = c * (E // NC)

    @pl.loop(0, ECH_T)
    def _(k):
        q = s + k * NS

        @pl.when(q < ECH_C)
        def _():
            pltpu.sync_copy(dst_hbm.at[pl.ds(ebase + q * CHUNK, CHUNK)], idx_v)
            pltpu.sync_copy(ones_v, acc_sh.at[idx_v], add=True)

    plsc.subcore_barrier()

    @pl.loop(0, PBLK // NS)
    def _(k):
        b = s + k * NS
        sl = pl.ds(b * CHUNK, CHUNK)
        pltpu.sync_copy(acc_sh.at[sl], out_hbm.at[c, 0, sl])


CPT = ECH_C // NS
NTAIL = ECH_C - CPT * NS

assert CPT % 2 == 0


@functools.partial(
    pl.kernel,
    out_type=jax.ShapeDtypeStruct((NC, N, D), jnp.float32),
    mesh=_mesh,
    scratch_types=[
        pltpu.VMEM((2, CHUNK), jnp.int32),
        pltpu.VMEM((2, CHUNK), jnp.int32),
        pltpu.VMEM((CHUNK, D), jnp.float32),
        pltpu.VMEM((CHUNK, D), jnp.float32),
        pltpu.VMEM_SHARED((N, D), jnp.float32),
        pltpu.SemaphoreType.DMA,
        pltpu.SemaphoreType.DMA,
    ],
)
def _aggregate_kernel(g_hbm, ei3_hbm, out_hbm, eiv0, eiv1, rows0, rows1,
                      acc_sh, sem0, sem1):
    c = lax.axis_index("c")
    s = lax.axis_index("s")

    _fill_zero_rows(rows0)

    @pl.loop(0, (RBLOCKS + NS - 1) // NS)
    def _(k):
        b = s + k * NS

        @pl.when(b < RBLOCKS)
        def _():
            pltpu.sync_copy(rows0.at[pl.ds(0, RBM)],
                            acc_sh.at[pl.ds(b * RBM, RBM)])

    plsc.subcore_barrier()

    cbase = c * ECH_C
    qb = cbase + s * CPT

    @pl.loop(0, CPT // 2)
    def _(k):
        q = qb + 2 * k
        pltpu.sync_copy(ei3_hbm.at[q], eiv0)
        d0 = pltpu.async_copy(g_hbm.at[eiv0.at[0]], rows0, sem0)
        pltpu.sync_copy(ei3_hbm.at[q + 1], eiv1)
        d1 = pltpu.async_copy(g_hbm.at[eiv1.at[0]], rows1, sem1)
        d0.wait()
        pltpu.sync_copy(rows0, acc_sh.at[eiv0.at[1]], add=True)
        d1.wait()
        pltpu.sync_copy(rows1, acc_sh.at[eiv1.at[1]], add=True)

    @pl.when(s < NTAIL)
    def _():
        pltpu.sync_copy(ei3_hbm.at[cbase + CPT * NS + s], eiv0)
        d0 = pltpu.async_copy(g_hbm.at[eiv0.at[0]], rows0, sem0)
        d0.wait()
        pltpu.sync_copy(rows0, acc_sh.at[eiv0.at[1]], add=True)

    plsc.subcore_barrier()

    @pl.loop(0, (RBLOCKS + NS - 1) // NS)
    def _(k):
        b = s + k * NS

        @pl.when(b < RBLOCKS)
        def _():
            sl = pl.ds(b * RBM, RBM)
            pltpu.sync_copy(acc_sh.at[sl], out_hbm.at[c, sl])


BM = 2000
GRID = N // BM


def _mm_scale_body(x_ref, w_ref, dinv_ref, h_ref, g_ref):
    h = jnp.dot(x_ref[...], w_ref[...], preferred_element_type=jnp.float32)
    h_ref[...] = h
    g_ref[...] = h * dinv_ref[...]


_mm_scale = pl.pallas_call(
    _mm_scale_body,
    grid=(GRID,),
    in_specs=[
        pl.BlockSpec((BM, D), lambda i: (i, 0)),
        pl.BlockSpec((D, D), lambda i: (0, 0)),
        pl.BlockSpec((BM, 1), lambda i: (i, 0)),
    ],
    out_specs=[
        pl.BlockSpec((BM, D), lambda i: (i, 0)),
        pl.BlockSpec((BM, D), lambda i: (i, 0)),
    ],
    out_shape=[
        jax.ShapeDtypeStruct((N, D), jnp.float32),
        jax.ShapeDtypeStruct((N, D), jnp.float32),
    ],
)


def _combine_mm_body(agg_ref, h_ref, dinv_ref, b_ref, w_ref, h2_ref, g2_ref):
    dinv = dinv_ref[...]
    tot = agg_ref[0] + agg_ref[1]
    out1 = jnp.maximum(dinv * tot + (dinv * dinv) * h_ref[...] + b_ref[...], 0.0)
    h2 = jnp.dot(out1, w_ref[...], preferred_element_type=jnp.float32)
    h2_ref[...] = h2
    g2_ref[...] = h2 * dinv


_combine_mm = pl.pallas_call(
    _combine_mm_body,
    grid=(GRID,),
    in_specs=[
        pl.BlockSpec((NC, BM, D), lambda i: (0, i, 0)),
        pl.BlockSpec((BM, D), lambda i: (i, 0)),
        pl.BlockSpec((BM, 1), lambda i: (i, 0)),
        pl.BlockSpec((1, D), lambda i: (0, 0)),
        pl.BlockSpec((D, D), lambda i: (0, 0)),
    ],
    out_specs=[
        pl.BlockSpec((BM, D), lambda i: (i, 0)),
        pl.BlockSpec((BM, D), lambda i: (i, 0)),
    ],
    out_shape=[
        jax.ShapeDtypeStruct((N, D), jnp.float32),
        jax.ShapeDtypeStruct((N, D), jnp.float32),
    ],
)


def _combine_body(agg_ref, h_ref, dinv_ref, b_ref, out_ref):
    dinv = dinv_ref[...]
    tot = agg_ref[0] + agg_ref[1]
    out_ref[...] = dinv * tot + (dinv * dinv) * h_ref[...] + b_ref[...]


_combine = pl.pallas_call(
    _combine_body,
    grid=(GRID,),
    in_specs=[
        pl.BlockSpec((NC, BM, D), lambda i: (0, i, 0)),
        pl.BlockSpec((BM, D), lambda i: (i, 0)),
        pl.BlockSpec((BM, 1), lambda i: (i, 0)),
        pl.BlockSpec((1, D), lambda i: (0, 0)),
    ],
    out_specs=pl.BlockSpec((BM, D), lambda i: (i, 0)),
    out_shape=jax.ShapeDtypeStruct((N, D), jnp.float32),
)


def kernel(x, edge_index, W1, b1, W2, b2):
    dst = edge_index[1]
    ei3 = jnp.transpose(edge_index.reshape(2, ECH, CHUNK), (1, 0, 2))

    deg_part = _degree_kernel(dst)
    deg = deg_part[0, 0, :N] + deg_part[1, 0, :N] + 1.0
    dinv = lax.rsqrt(jnp.maximum(deg, 1.0))[:, None]

    h1, g1 = _mm_scale(x, W1, dinv)
    agg1 = _aggregate_kernel(g1, ei3)
    h2, g2 = _combine_mm(agg1, h1, dinv, b1[None, :], W2)
    agg2 = _aggregate_kernel(g2, ei3)
    out = _combine(agg2, h2, dinv, b2[None, :])
    return out


# speedup vs baseline: 27.9026x; 1.1651x over previous
import functools

import jax
import jax.numpy as jnp
from jax import lax
from jax.experimental import pallas as pl
from jax.experimental.pallas import tpu as pltpu
from jax.experimental.pallas import tpu_sc as plsc

N = 10000
E = 320000
D = 128

NC = 2
NS = 16
NW = NC * NS

CHUNK = 128
ECH = E // CHUNK
ECH_C = ECH // NC
ECH_T = (ECH_C + NS - 1) // NS
NP = 10240
PBLK = NP // CHUNK
RBM = 80
RBLOCKS = N // RBM
CPT = ECH_C // NS
NTAIL = ECH_C - CPT * NS

assert ECH * CHUNK == E and ECH_C * NC == ECH and RBLOCKS * RBM == N
assert CPT % 2 == 0

_mesh = plsc.VectorSubcoreMesh(core_axis_name="c", subcore_axis_name="s")


def _fill_zero_rows(buf):
    @pl.loop(0, CHUNK)
    def _(i):
        for j in range(D // 16):
            buf[i, pl.ds(j * 16, 16)] = jnp.zeros((16,), jnp.float32)


@functools.partial(
    pl.kernel,
    out_type=jax.ShapeDtypeStruct((NC, 1, NP), jnp.float32),
    mesh=_mesh,
    scratch_types=[
        pltpu.VMEM((CHUNK,), jnp.int32),
        pltpu.VMEM((CHUNK,), jnp.float32),
        pltpu.VMEM((CHUNK,), jnp.float32),
        pltpu.VMEM_SHARED((NP,), jnp.float32),
    ],
)
def _degree_kernel(dst_hbm, out_hbm, idx_v, ones_v, zero_v, acc_sh):
    c = lax.axis_index("c")
    s = lax.axis_index("s")
    for k in range(CHUNK // 16):
        ones_v[pl.ds(k * 16, 16)] = jnp.ones((16,), jnp.float32)
        zero_v[pl.ds(k * 16, 16)] = jnp.zeros((16,), jnp.float32)

    @pl.loop(0, PBLK // NS)
    def _(k):
        b = s + k * NS
        pltpu.sync_copy(zero_v, acc_sh.at[pl.ds(b * CHUNK, CHUNK)])

    plsc.subcore_barrier()

    ebase = c * (E // NC)

    @pl.loop(0, ECH_T)
    def _(k):
        q = s + k * NS

        @pl.when(q < ECH_C)
        def _():
            pltpu.sync_copy(dst_hbm.at[pl.ds(ebase + q * CHUNK, CHUNK)], idx_v)
            pltpu.sync_copy(ones_v, acc_sh.at[idx_v], add=True)

    plsc.subcore_barrier()

    @pl.loop(0, PBLK // NS)
    def _(k):
        b = s + k * NS
        sl = pl.ds(b * CHUNK, CHUNK)
        pltpu.sync_copy(acc_sh.at[sl], out_hbm.at[c, 0, sl])


@functools.partial(
    pl.kernel,
    out_type=jax.ShapeDtypeStruct((NC, N, D), jnp.float32),
    mesh=_mesh,
    scratch_types=[
        pltpu.VMEM((2, CHUNK), jnp.int32),
        pltpu.VMEM((2, CHUNK), jnp.int32),
        pltpu.VMEM((CHUNK, D), jnp.float32),
        pltpu.VMEM((CHUNK, D), jnp.float32),
        pltpu.VMEM_SHARED((N, D), jnp.float32),
        pltpu.SemaphoreType.DMA,
        pltpu.SemaphoreType.DMA,
    ],
)
def _aggregate_kernel(g_hbm, ei3_hbm, out_hbm, eiv0, eiv1, rows0, rows1,
                      acc_sh, sem0, sem1):
    c = lax.axis_index("c")
    s = lax.axis_index("s")
    eiv = (eiv0, eiv1)
    rows = (rows0, rows1)
    sems = (sem0, sem1)

    _fill_zero_rows(rows0)

    @pl.loop(0, (RBLOCKS + NS - 1) // NS)
    def _(k):
        b = s + k * NS

        @pl.when(b < RBLOCKS)
        def _():
            pltpu.sync_copy(rows0.at[pl.ds(0, RBM)],
                            acc_sh.at[pl.ds(b * RBM, RBM)])

    plsc.subcore_barrier()

    cbase = c * ECH_C
    qb = cbase + s * CPT

    def fire(q, b):
        pltpu.sync_copy(ei3_hbm.at[q], eiv[b])
        pltpu.async_copy(g_hbm.at[eiv[b].at[0]], rows[b], sems[b])

    def wait_gather(b):
        pltpu.make_async_copy(g_hbm.at[eiv[b].at[0]], rows[b], sems[b]).wait()

    def scatter(b):
        pltpu.sync_copy(rows[b], acc_sh.at[eiv[b].at[1]], add=True)

    fire(qb, 0)

    @pl.loop(0, CPT, step=2)
    def _(k):
        q = qb + k
        fire(q + 1, 1)
        wait_gather(0)
        scatter(0)

        @pl.when(k + 2 < CPT)
        def _():
            fire(q + 2, 0)

        wait_gather(1)
        scatter(1)

    @pl.when(s < NTAIL)
    def _():
        fire(cbase + CPT * NS + s, 0)
        wait_gather(0)
        scatter(0)

    plsc.subcore_barrier()

    @pl.loop(0, (RBLOCKS + NS - 1) // NS)
    def _(k):
        b = s + k * NS

        @pl.when(b < RBLOCKS)
        def _():
            sl = pl.ds(b * RBM, RBM)
            pltpu.sync_copy(acc_sh.at[sl], out_hbm.at[c, sl])


BM = 2000
GRID = N // BM


def _mm_scale_body(x_ref, w_ref, dinv_ref, h_ref, g_ref):
    h = jnp.dot(x_ref[...], w_ref[...], preferred_element_type=jnp.float32)
    h_ref[...] = h
    g_ref[...] = h * dinv_ref[...]


_mm_scale = pl.pallas_call(
    _mm_scale_body,
    grid=(GRID,),
    in_specs=[
        pl.BlockSpec((BM, D), lambda i: (i, 0)),
        pl.BlockSpec((D, D), lambda i: (0, 0)),
        pl.BlockSpec((BM, 1), lambda i: (i, 0)),
    ],
    out_specs=[
        pl.BlockSpec((BM, D), lambda i: (i, 0)),
        pl.BlockSpec((BM, D), lambda i: (i, 0)),
    ],
    out_shape=[
        jax.ShapeDtypeStruct((N, D), jnp.float32),
        jax.ShapeDtypeStruct((N, D), jnp.float32),
    ],
)


def _combine_mm_body(agg_ref, h_ref, dinv_ref, b_ref, w_ref, h2_ref, g2_ref):
    dinv = dinv_ref[...]
    tot = agg_ref[0] + agg_ref[1]
    out1 = jnp.maximum(dinv * tot + (dinv * dinv) * h_ref[...] + b_ref[...], 0.0)
    h2 = jnp.dot(out1, w_ref[...], preferred_element_type=jnp.float32)
    h2_ref[...] = h2
    g2_ref[...] = h2 * dinv


_combine_mm = pl.pallas_call(
    _combine_mm_body,
    grid=(GRID,),
    in_specs=[
        pl.BlockSpec((NC, BM, D), lambda i: (0, i, 0)),
        pl.BlockSpec((BM, D), lambda i: (i, 0)),
        pl.BlockSpec((BM, 1), lambda i: (i, 0)),
        pl.BlockSpec((1, D), lambda i: (0, 0)),
        pl.BlockSpec((D, D), lambda i: (0, 0)),
    ],
    out_specs=[
        pl.BlockSpec((BM, D), lambda i: (i, 0)),
        pl.BlockSpec((BM, D), lambda i: (i, 0)),
    ],
    out_shape=[
        jax.ShapeDtypeStruct((N, D), jnp.float32),
        jax.ShapeDtypeStruct((N, D), jnp.float32),
    ],
)


def _combine_body(agg_ref, h_ref, dinv_ref, b_ref, out_ref):
    dinv = dinv_ref[...]
    tot = agg_ref[0] + agg_ref[1]
    out_ref[...] = dinv * tot + (dinv * dinv) * h_ref[...] + b_ref[...]


_combine = pl.pallas_call(
    _combine_body,
    grid=(GRID,),
    in_specs=[
        pl.BlockSpec((NC, BM, D), lambda i: (0, i, 0)),
        pl.BlockSpec((BM, D), lambda i: (i, 0)),
        pl.BlockSpec((BM, 1), lambda i: (i, 0)),
        pl.BlockSpec((1, D), lambda i: (0, 0)),
    ],
    out_specs=pl.BlockSpec((BM, D), lambda i: (i, 0)),
    out_shape=jax.ShapeDtypeStruct((N, D), jnp.float32),
)


def kernel(x, edge_index, W1, b1, W2, b2):
    dst = edge_index[1]
    ei3 = jnp.transpose(edge_index.reshape(2, ECH, CHUNK), (1, 0, 2))

    deg_part = _degree_kernel(dst)
    deg = deg_part[0, 0, :N] + deg_part[1, 0, :N] + 1.0
    dinv = lax.rsqrt(jnp.maximum(deg, 1.0))[:, None]

    h1, g1 = _mm_scale(x, W1, dinv)
    agg1 = _aggregate_kernel(g1, ei3)
    h2, g2 = _combine_mm(agg1, h1, dinv, b1[None, :], W2)
    agg2 = _aggregate_kernel(g2, ei3)
    out = _combine(agg2, h2, dinv, b2[None, :])
    return out
